# Initial kernel scaffold; baseline (speedup 1.0000x reference)
import jax
import jax.numpy as jnp
from jax.experimental import pallas as pl


def kernel(x, logits, W, b):
    raise NotImplementedError("write your pallas kernel here")


# speedup vs baseline: 2.2239x; 2.2239x over previous
import functools

import jax
import jax.numpy as jnp
from jax import lax
from jax.experimental import pallas as pl
from jax.experimental.pallas import tpu as pltpu
from jax.experimental.pallas import tpu_sc as plsc

E = 8
K = 2
D = 1024
N = 8192
BN = 512


def _topk2_ids_sc(logits16):
    mesh = plsc.VectorSubcoreMesh(core_axis_name="c", subcore_axis_name="s")

    @functools.partial(
        pl.kernel,
        out_type=jax.ShapeDtypeStruct((16,), jnp.int32),
        mesh=mesh,
        scratch_types=[
            pltpu.VMEM((16,), jnp.float32),
            pltpu.VMEM((16,), jnp.int32),
        ],
    )
    def topk_kernel(logits_hbm, ids_hbm, logits_v, ids_v):
        cid = lax.axis_index("c")
        sid = lax.axis_index("s")

        @pl.when(jnp.logical_and(cid == 0, sid == 0))
        def _():
            pltpu.sync_copy(logits_hbm, logits_v)
            keys = logits_v[...]
            neg_inf = jnp.float32(-jnp.inf)
            m1, i1 = neg_inf, jnp.int32(0)
            m2, i2 = neg_inf, jnp.int32(0)
            for j in range(E):
                v = keys[j]
                gt1 = v > m1
                gt2 = v > m2
                n_m2 = jnp.where(gt1, m1, jnp.where(gt2, v, m2))
                n_i2 = jnp.where(gt1, i1, jnp.where(gt2, jnp.int32(j), i2))
                m1 = jnp.where(gt1, v, m1)
                i1 = jnp.where(gt1, jnp.int32(j), i1)
                m2, i2 = n_m2, n_i2
            iota = lax.iota(jnp.int32, 16)
            ids_v[...] = jnp.where(iota == 0, i1,
                                   jnp.where(iota == 1, i2, 0))
            pltpu.sync_copy(ids_v, ids_hbm)

    return topk_kernel(logits16)


def _mixed_op_tc(ids, x, W, b):
    b3 = b.reshape(E, 1, D)

    def body(ids_ref, x_ref, w0_ref, w1_ref, b0_ref, b1_ref, o_ref):
        xv = x_ref[...]
        r0 = jnp.dot(xv, w0_ref[0], preferred_element_type=jnp.float32)
        r1 = jnp.dot(xv, w1_ref[0], preferred_element_type=jnp.float32)
        r0 = jnp.maximum(r0 + b0_ref[0, 0], 0.0)
        r1 = jnp.maximum(r1 + b1_ref[0, 0], 0.0)
        o_ref[...] = r0 + r1

    grid_spec = pltpu.PrefetchScalarGridSpec(
        num_scalar_prefetch=1,
        grid=(N // BN,),
        in_specs=[
            pl.BlockSpec((BN, D), lambda i, ids: (i, 0)),
            pl.BlockSpec((1, D, D), lambda i, ids: (ids[0], 0, 0)),
            pl.BlockSpec((1, D, D), lambda i, ids: (ids[1], 0, 0)),
            pl.BlockSpec((1, 1, D), lambda i, ids: (ids[0], 0, 0)),
            pl.BlockSpec((1, 1, D), lambda i, ids: (ids[1], 0, 0)),
        ],
        out_specs=pl.BlockSpec((BN, D), lambda i, ids: (i, 0)),
    )
    return pl.pallas_call(
        body,
        grid_spec=grid_spec,
        out_shape=jax.ShapeDtypeStruct((N, D), jnp.float32),
    )(ids, x, W, W, b3, b3)


def kernel(x, logits, W, b):
    logits16 = jnp.full((16,), -jnp.inf, dtype=jnp.float32).at[:E].set(logits)
    ids = _topk2_ids_sc(logits16)
    return _mixed_op_tc(ids, x, W, b)


# speedup vs baseline: 2.2423x; 1.0083x over previous
import functools

import jax
import jax.numpy as jnp
from jax import lax
from jax.experimental import pallas as pl
from jax.experimental.pallas import tpu as pltpu
from jax.experimental.pallas import tpu_sc as plsc

E = 8
K = 2
D = 1024
N = 8192
BN = 512


def _topk2_ids_sc(logits16):
    mesh = plsc.VectorSubcoreMesh(core_axis_name="c", subcore_axis_name="s")

    @functools.partial(
        pl.kernel,
        out_type=jax.ShapeDtypeStruct((16,), jnp.int32),
        mesh=mesh,
        scratch_types=[
            pltpu.VMEM((16,), jnp.float32),
            pltpu.VMEM((16,), jnp.int32),
        ],
    )
    def topk_kernel(logits_hbm, ids_hbm, logits_v, ids_v):
        cid = lax.axis_index("c")
        sid = lax.axis_index("s")

        @pl.when(jnp.logical_and(cid == 0, sid == 0))
        def _():
            pltpu.sync_copy(logits_hbm, logits_v)
            keys = logits_v[...]
            neg_inf = jnp.float32(-jnp.inf)
            m1, i1 = neg_inf, jnp.int32(0)
            m2, i2 = neg_inf, jnp.int32(0)
            for j in range(E):
                v = keys[j]
                gt1 = v > m1
                gt2 = v > m2
                n_m2 = jnp.where(gt1, m1, jnp.where(gt2, v, m2))
                n_i2 = jnp.where(gt1, i1, jnp.where(gt2, jnp.int32(j), i2))
                m1 = jnp.where(gt1, v, m1)
                i1 = jnp.where(gt1, jnp.int32(j), i1)
                m2, i2 = n_m2, n_i2
            iota = lax.iota(jnp.int32, 16)
            ids_v[...] = jnp.where(iota == 0, i1,
                                   jnp.where(iota == 1, i2, 0))
            pltpu.sync_copy(ids_v, ids_hbm)

    return topk_kernel(logits16)


def _mixed_op_tc(ids, x, W, b):
    b3 = b.reshape(E, 1, D)

    def body(ids_ref, x_ref, w0_ref, w1_ref, b0_ref, b1_ref, o_ref):
        xv = x_ref[...].astype(jnp.bfloat16)
        w0 = w0_ref[0].astype(jnp.bfloat16)
        w1 = w1_ref[0].astype(jnp.bfloat16)
        r0 = jnp.dot(xv, w0, preferred_element_type=jnp.float32)
        r1 = jnp.dot(xv, w1, preferred_element_type=jnp.float32)
        r0 = jnp.maximum(r0 + b0_ref[0, 0], 0.0)
        r1 = jnp.maximum(r1 + b1_ref[0, 0], 0.0)
        o_ref[...] = r0 + r1

    grid_spec = pltpu.PrefetchScalarGridSpec(
        num_scalar_prefetch=1,
        grid=(N // BN,),
        in_specs=[
            pl.BlockSpec((BN, D), lambda i, ids: (i, 0)),
            pl.BlockSpec((1, D, D), lambda i, ids: (ids[0], 0, 0)),
            pl.BlockSpec((1, D, D), lambda i, ids: (ids[1], 0, 0)),
            pl.BlockSpec((1, 1, D), lambda i, ids: (ids[0], 0, 0)),
            pl.BlockSpec((1, 1, D), lambda i, ids: (ids[1], 0, 0)),
        ],
        out_specs=pl.BlockSpec((BN, D), lambda i, ids: (i, 0)),
    )
    return pl.pallas_call(
        body,
        grid_spec=grid_spec,
        out_shape=jax.ShapeDtypeStruct((N, D), jnp.float32),
    )(ids, x, W, W, b3, b3)


def kernel(x, logits, W, b):
    logits16 = jnp.full((16,), -jnp.inf, dtype=jnp.float32).at[:E].set(logits)
    ids = _topk2_ids_sc(logits16)
    return _mixed_op_tc(ids, x, W, b)


# speedup vs baseline: 2.2735x; 1.0139x over previous
import functools

import jax
import jax.numpy as jnp
from jax import lax
from jax.experimental import pallas as pl
from jax.experimental.pallas import tpu as pltpu
from jax.experimental.pallas import tpu_sc as plsc

E = 8
K = 2
D = 1024
N = 8192
BN = 1024


def _topk2_ids_sc(logits16):
    mesh = plsc.VectorSubcoreMesh(core_axis_name="c", subcore_axis_name="s")

    @functools.partial(
        pl.kernel,
        out_type=jax.ShapeDtypeStruct((16,), jnp.int32),
        mesh=mesh,
        scratch_types=[
            pltpu.VMEM((16,), jnp.float32),
            pltpu.VMEM((16,), jnp.int32),
        ],
    )
    def topk_kernel(logits_hbm, ids_hbm, logits_v, ids_v):
        cid = lax.axis_index("c")
        sid = lax.axis_index("s")

        @pl.when(jnp.logical_and(cid == 0, sid == 0))
        def _():
            pltpu.sync_copy(logits_hbm, logits_v)
            keys = logits_v[...]
            neg_inf = jnp.float32(-jnp.inf)
            m1, i1 = neg_inf, jnp.int32(0)
            m2, i2 = neg_inf, jnp.int32(0)
            for j in range(E):
                v = keys[j]
                gt1 = v > m1
                gt2 = v > m2
                n_m2 = jnp.where(gt1, m1, jnp.where(gt2, v, m2))
                n_i2 = jnp.where(gt1, i1, jnp.where(gt2, jnp.int32(j), i2))
                m1 = jnp.where(gt1, v, m1)
                i1 = jnp.where(gt1, jnp.int32(j), i1)
                m2, i2 = n_m2, n_i2
            iota = lax.iota(jnp.int32, 16)
            ids_v[...] = jnp.where(iota == 0, i1,
                                   jnp.where(iota == 1, i2, 0))
            pltpu.sync_copy(ids_v, ids_hbm)

    return topk_kernel(logits16)


def _mixed_op_tc(ids, x, W, b):
    b3 = b.reshape(E, 1, D)

    def body(ids_ref, x_ref, w0_ref, w1_ref, b0_ref, b1_ref, o_ref):
        xv = x_ref[...].astype(jnp.bfloat16)
        w0 = w0_ref[0].astype(jnp.bfloat16)
        w1 = w1_ref[0].astype(jnp.bfloat16)
        r0 = jnp.dot(xv, w0, preferred_element_type=jnp.float32)
        r1 = jnp.dot(xv, w1, preferred_element_type=jnp.float32)
        r0 = jnp.maximum(r0 + b0_ref[0, 0], 0.0)
        r1 = jnp.maximum(r1 + b1_ref[0, 0], 0.0)
        o_ref[...] = r0 + r1

    grid_spec = pltpu.PrefetchScalarGridSpec(
        num_scalar_prefetch=1,
        grid=(N // BN,),
        in_specs=[
            pl.BlockSpec((BN, D), lambda i, ids: (i, 0)),
            pl.BlockSpec((1, D, D), lambda i, ids: (ids[0], 0, 0)),
            pl.BlockSpec((1, D, D), lambda i, ids: (ids[1], 0, 0)),
            pl.BlockSpec((1, 1, D), lambda i, ids: (ids[0], 0, 0)),
            pl.BlockSpec((1, 1, D), lambda i, ids: (ids[1], 0, 0)),
        ],
        out_specs=pl.BlockSpec((BN, D), lambda i, ids: (i, 0)),
    )
    return pl.pallas_call(
        body,
        grid_spec=grid_spec,
        out_shape=jax.ShapeDtypeStruct((N, D), jnp.float32),
    )(ids, x, W, W, b3, b3)


def kernel(x, logits, W, b):
    logits16 = jnp.full((16,), -jnp.inf, dtype=jnp.float32).at[:E].set(logits)
    ids = _topk2_ids_sc(logits16)
    return _mixed_op_tc(ids, x, W, b)
